# baseline (device time: 38869 ns/iter reference)
import jax
import jax.numpy as jnp
from jax import lax
from jax.experimental import pallas as pl
from jax.experimental.pallas import tpu as pltpu

_MESH = pl.DeviceIdType.MESH


def kernel(x, dy):
    m, d = x.shape
    _, f = dy.shape
    out_rows = d // 2
    fq = f // 4

    def body(x_ref, dy_ref, out_ref, xpart_ref, comm_ref,
             x_sems, send_sems, recv_sems):
        X = lax.axis_index("x")
        Y = lax.axis_index("y")
        Z = lax.axis_index("z")
        r = 2 * Y + jnp.bitwise_xor(Y, Z)

        barrier = pltpu.get_barrier_semaphore()
        for nbr in ((1 - X, Y, Z), (X, 1 - Y, Z), (X, Y, 1 - Z)):
            pl.semaphore_signal(barrier, inc=1, device_id=nbr,
                                device_id_type=_MESH)
        pl.semaphore_wait(barrier, 3)

        dy_sl = dy_ref[:, pl.ds(r * fq, fq)]
        dims = (((0,), (0,)), ((), ()))
        x_other = x_ref[:, pl.ds((1 - X) * out_rows, out_rows)]
        xpart_ref[0] = lax.dot_general(x_other, dy_sl, dims,
                                       preferred_element_type=jnp.float32)
        xch = pltpu.make_async_remote_copy(
            src_ref=xpart_ref.at[0], dst_ref=xpart_ref.at[1],
            send_sem=x_sems.at[0], recv_sem=x_sems.at[1],
            device_id=(1 - X, Y, Z), device_id_type=_MESH)
        xch.start()
        x_own = x_ref[:, pl.ds(X * out_rows, out_rows)]
        p_own = lax.dot_general(x_own, dy_sl, dims,
                                preferred_element_type=jnp.float32)
        xch.wait()
        quarter = p_own + xpart_ref[1]
        out_ref[:, pl.ds(r * fq, fq)] = quarter
        comm_ref[0] = quarter

        rp = jnp.remainder(r + 1, 4)
        ry = rp // 2
        rz = jnp.bitwise_xor(ry, jnp.remainder(rp, 2))
        for h in range(3):
            s = h % 2
            rc = (h + 1) % 2
            rd = pltpu.make_async_remote_copy(
                src_ref=comm_ref.at[s], dst_ref=comm_ref.at[rc],
                send_sem=send_sems.at[s], recv_sem=recv_sems.at[rc],
                device_id=(X, ry, rz), device_id_type=_MESH)
            rd.start()
            rd.wait()
            origin = jnp.remainder(r - h - 1, 4)
            out_ref[:, pl.ds(origin * fq, fq)] = comm_ref[rc]

    return pl.pallas_call(
        body,
        out_shape=jax.ShapeDtypeStruct((out_rows, f), jnp.float32),
        in_specs=[pl.BlockSpec(memory_space=pltpu.VMEM),
                  pl.BlockSpec(memory_space=pltpu.VMEM)],
        out_specs=pl.BlockSpec(memory_space=pltpu.VMEM),
        scratch_shapes=[
            pltpu.VMEM((2, out_rows, fq), jnp.float32),
            pltpu.VMEM((2, out_rows, fq), jnp.float32),
            pltpu.SemaphoreType.DMA((2,)),
            pltpu.SemaphoreType.DMA((2,)),
            pltpu.SemaphoreType.DMA((2,)),
        ],
        compiler_params=pltpu.CompilerParams(collective_id=0),
    )(x, dy)


# device time: 24242 ns/iter; 1.6034x vs baseline; 1.6034x over previous
import jax
import jax.numpy as jnp
from jax import lax
from jax.experimental import pallas as pl
from jax.experimental.pallas import tpu as pltpu

_MESH = pl.DeviceIdType.MESH
_C = 2


def kernel(x, dy):
    m, d = x.shape
    _, f = dy.shape
    out_rows = d // 2
    fq = f // 4
    w = fq // _C
    hw = w // 2

    def body(x_ref, dy_ref, out_ref, xpart_ref,
             xs_sems, xr_sems, ssems, rsems):
        X = lax.axis_index("x")
        Y = lax.axis_index("y")
        Z = lax.axis_index("z")
        r = 2 * Y + jnp.bitwise_xor(Y, Z)
        lq = jnp.remainder(r + 3, 4)
        rq = jnp.remainder(r + 1, 4)
        dq = jnp.remainder(r + 2, 4)

        def coords(p):
            py = p // 2
            return py, jnp.bitwise_xor(py, jnp.remainder(p, 2))

        ly, lz = coords(lq)
        ry, rz = coords(rq)
        left = (X, ly, lz)
        right = (X, ry, rz)
        xpeer = (1 - X, Y, Z)

        barrier = pltpu.get_barrier_semaphore()
        for nbr in (xpeer, (X, 1 - Y, Z), (X, Y, 1 - Z)):
            pl.semaphore_signal(barrier, inc=1, device_id=nbr,
                                device_id_type=_MESH)
        pl.semaphore_wait(barrier, 3)

        dims = (((0,), (0,)), ((), ()))
        x_own = x_ref[:, pl.ds(X * out_rows, out_rows)]
        x_other = x_ref[:, pl.ds((1 - X) * out_rows, out_rows)]

        xchs = []
        for c in range(_C):
            dy_c = dy_ref[:, pl.ds(r * fq + c * w, w)]
            xpart_ref[0, c] = lax.dot_general(
                x_other, dy_c, dims, preferred_element_type=jnp.float32)
            xch = pltpu.make_async_remote_copy(
                src_ref=xpart_ref.at[0, c], dst_ref=xpart_ref.at[1, c],
                send_sem=xs_sems.at[c], recv_sem=xr_sems.at[c],
                device_id=xpeer, device_id_type=_MESH)
            xch.start()
            xchs.append(xch)

        def copy(src_cols, dst_cols, width, dev, send_slot, recv_slot, c):
            return pltpu.make_async_remote_copy(
                src_ref=out_ref.at[:, pl.ds(src_cols, width)],
                dst_ref=out_ref.at[:, pl.ds(dst_cols, width)],
                send_sem=ssems.at[send_slot, c],
                recv_sem=rsems.at[recv_slot, c],
                device_id=dev, device_id_type=_MESH)

        senders = []
        a_in_l, a_in_r = [], []
        b_in = []
        for c in range(_C):
            own = r * fq + c * w
            dy_c = dy_ref[:, pl.ds(own, w)]
            p_own = lax.dot_general(
                x_own, dy_c, dims, preferred_element_type=jnp.float32)
            xchs[c].wait_recv()
            out_ref[:, pl.ds(own, w)] = p_own + xpart_ref[1, c]
            a_l = copy(own, own, w, left, 0, 1, c)
            a_r = copy(own, own, w, right, 1, 0, c)
            a_l.start()
            a_r.start()
            senders += [a_l, a_r]
            a_in_l.append(copy(lq * fq + c * w, lq * fq + c * w, w,
                               left, 0, 0, c))
            a_in_r.append(copy(rq * fq + c * w, rq * fq + c * w, w,
                               right, 1, 1, c))

        for c in range(_C):
            a_in_l[c].wait_recv()
            b_r = copy(lq * fq + c * w, lq * fq + c * w, hw, right, 2, 2, c)
            b_r.start()
            senders.append(b_r)
            a_in_r[c].wait_recv()
            b_l = copy(rq * fq + c * w + hw, rq * fq + c * w + hw, hw,
                       left, 3, 3, c)
            b_l.start()
            senders.append(b_l)
            b_in.append(copy(dq * fq + c * w, dq * fq + c * w, hw,
                             left, 2, 2, c))
            b_in.append(copy(dq * fq + c * w + hw, dq * fq + c * w + hw, hw,
                             right, 3, 3, c))

        for rcv in b_in:
            rcv.wait_recv()
        for snd in senders:
            snd.wait_send()
        for xch in xchs:
            xch.wait_send()

    return pl.pallas_call(
        body,
        out_shape=jax.ShapeDtypeStruct((out_rows, f), jnp.float32),
        in_specs=[pl.BlockSpec(memory_space=pltpu.VMEM),
                  pl.BlockSpec(memory_space=pltpu.VMEM)],
        out_specs=pl.BlockSpec(memory_space=pltpu.VMEM),
        scratch_shapes=[
            pltpu.VMEM((2, _C, out_rows, w), jnp.float32),
            pltpu.SemaphoreType.DMA((_C,)),
            pltpu.SemaphoreType.DMA((_C,)),
            pltpu.SemaphoreType.DMA((4, _C)),
            pltpu.SemaphoreType.DMA((4, _C)),
        ],
        compiler_params=pltpu.CompilerParams(collective_id=0),
    )(x, dy)


# device time: 22974 ns/iter; 1.6919x vs baseline; 1.0552x over previous
import jax
import jax.numpy as jnp
from jax import lax
from jax.experimental import pallas as pl
from jax.experimental.pallas import tpu as pltpu

_MESH = pl.DeviceIdType.MESH
_C = 4


def kernel(x, dy):
    m, d = x.shape
    _, f = dy.shape
    out_rows = d // 2
    hr = out_rows // 2
    fq = f // 4
    w = fq // _C

    def body(x_ref, dy_ref, out_ref, part_ref, xrecv_ref,
             xs_sems, xr_sems, ssems, rsems):
        X = lax.axis_index("x")
        Y = lax.axis_index("y")
        Z = lax.axis_index("z")
        r = 2 * Y + jnp.bitwise_xor(Y, Z)
        lq = jnp.remainder(r + 3, 4)
        rq = jnp.remainder(r + 1, 4)
        dq = jnp.remainder(r + 2, 4)

        def coords(p):
            py = p // 2
            return py, jnp.bitwise_xor(py, jnp.remainder(p, 2))

        ly, lz = coords(lq)
        ry, rz = coords(rq)
        left = (X, ly, lz)
        right = (X, ry, rz)
        xpeer = (1 - X, Y, Z)

        barrier = pltpu.get_barrier_semaphore()
        for nbr in (xpeer, (X, 1 - Y, Z), (X, Y, 1 - Z)):
            pl.semaphore_signal(barrier, inc=1, device_id=nbr,
                                device_id_type=_MESH)
        pl.semaphore_wait(barrier, 3)

        dy_sl = dy_ref[:, pl.ds(r * fq, fq)]
        part_ref[...] = lax.dot_general(
            x_ref[...], dy_sl, (((0,), (0,)), ((), ())),
            preferred_element_type=jnp.float32)
        own_r0 = X * out_rows
        oth_r0 = (1 - X) * out_rows

        xchs = []
        for c in range(_C):
            xch = pltpu.make_async_remote_copy(
                src_ref=part_ref.at[pl.ds(oth_r0, out_rows), pl.ds(c * w, w)],
                dst_ref=xrecv_ref.at[:, pl.ds(c * w, w)],
                send_sem=xs_sems.at[c], recv_sem=xr_sems.at[c],
                device_id=xpeer, device_id_type=_MESH)
            xch.start()
            xchs.append(xch)

        def copy(row0, nrows, cols, dev, send_slot, recv_slot, c):
            return pltpu.make_async_remote_copy(
                src_ref=out_ref.at[pl.ds(row0, nrows), pl.ds(cols, w)],
                dst_ref=out_ref.at[pl.ds(row0, nrows), pl.ds(cols, w)],
                send_sem=ssems.at[send_slot, c],
                recv_sem=rsems.at[recv_slot, c],
                device_id=dev, device_id_type=_MESH)

        senders = []
        a_in_l, a_in_r = [], []
        b_in = []
        for c in range(_C):
            own = r * fq + c * w
            xchs[c].wait_recv()
            out_ref[:, pl.ds(own, w)] = (
                part_ref[pl.ds(own_r0, out_rows), pl.ds(c * w, w)]
                + xrecv_ref[:, pl.ds(c * w, w)])
            a_l = copy(0, out_rows, own, left, 0, 1, c)
            a_r = copy(0, out_rows, own, right, 1, 0, c)
            a_l.start()
            a_r.start()
            senders += [a_l, a_r]
            a_in_l.append(copy(0, out_rows, lq * fq + c * w, left, 0, 0, c))
            a_in_r.append(copy(0, out_rows, rq * fq + c * w, right, 1, 1, c))

        for c in range(_C):
            a_in_l[c].wait_recv()
            b_r = copy(0, hr, lq * fq + c * w, right, 2, 2, c)
            b_r.start()
            senders.append(b_r)
            a_in_r[c].wait_recv()
            b_l = copy(hr, hr, rq * fq + c * w, left, 3, 3, c)
            b_l.start()
            senders.append(b_l)
            b_in.append(copy(0, hr, dq * fq + c * w, left, 2, 2, c))
            b_in.append(copy(hr, hr, dq * fq + c * w, right, 3, 3, c))

        for rcv in b_in:
            rcv.wait_recv()
        for snd in senders:
            snd.wait_send()
        for xch in xchs:
            xch.wait_send()

    return pl.pallas_call(
        body,
        out_shape=jax.ShapeDtypeStruct((out_rows, f), jnp.float32),
        in_specs=[pl.BlockSpec(memory_space=pltpu.VMEM),
                  pl.BlockSpec(memory_space=pltpu.VMEM)],
        out_specs=pl.BlockSpec(memory_space=pltpu.VMEM),
        scratch_shapes=[
            pltpu.VMEM((d, fq), jnp.float32),
            pltpu.VMEM((out_rows, fq), jnp.float32),
            pltpu.SemaphoreType.DMA((_C,)),
            pltpu.SemaphoreType.DMA((_C,)),
            pltpu.SemaphoreType.DMA((4, _C)),
            pltpu.SemaphoreType.DMA((4, _C)),
        ],
        compiler_params=pltpu.CompilerParams(collective_id=0),
    )(x, dy)


# device time: 22750 ns/iter; 1.7085x vs baseline; 1.0098x over previous
import jax
import jax.numpy as jnp
from jax import lax
from jax.experimental import pallas as pl
from jax.experimental.pallas import tpu as pltpu

_MESH = pl.DeviceIdType.MESH
_C = 4


def kernel(x, dy):
    m, d = x.shape
    _, f = dy.shape
    out_rows = d // 2
    hr = out_rows // 2
    fq = f // 4
    w = fq // _C

    def body(x_ref, dy_ref, out_ref, xsend_ref, xrecv_ref, comm_ref,
             xs_sems, xr_sems, ssems, rsems):
        X = lax.axis_index("x")
        Y = lax.axis_index("y")
        Z = lax.axis_index("z")
        r = 2 * Y + jnp.bitwise_xor(Y, Z)
        lq = jnp.remainder(r + 3, 4)
        rq = jnp.remainder(r + 1, 4)
        dq = jnp.remainder(r + 2, 4)

        def coords(p):
            py = p // 2
            return py, jnp.bitwise_xor(py, jnp.remainder(p, 2))

        ly, lz = coords(lq)
        ry, rz = coords(rq)
        left = (X, ly, lz)
        right = (X, ry, rz)
        xpeer = (1 - X, Y, Z)

        barrier = pltpu.get_barrier_semaphore()
        for nbr in (xpeer, (X, 1 - Y, Z), (X, Y, 1 - Z)):
            pl.semaphore_signal(barrier, inc=1, device_id=nbr,
                                device_id_type=_MESH)
        pl.semaphore_wait(barrier, 3)

        dims = (((0,), (0,)), ((), ()))
        dy_sl = dy_ref[:, pl.ds(r * fq, fq)]
        x_oth = x_ref[:, pl.ds((1 - X) * out_rows, out_rows)]
        p_oth = lax.dot_general(x_oth, dy_sl, dims,
                                preferred_element_type=jnp.float32)
        xchs = []
        for c in range(_C):
            xsend_ref[c] = p_oth[:, c * w:(c + 1) * w]
            xch = pltpu.make_async_remote_copy(
                src_ref=xsend_ref.at[c], dst_ref=xrecv_ref.at[c],
                send_sem=xs_sems.at[c], recv_sem=xr_sems.at[c],
                device_id=xpeer, device_id_type=_MESH)
            xch.start()
            xchs.append(xch)
        x_own = x_ref[:, pl.ds(X * out_rows, out_rows)]
        p_own = lax.dot_general(x_own, dy_sl, dims,
                                preferred_element_type=jnp.float32)

        def copy(src, dst, dev, send_slot, recv_slot, c):
            return pltpu.make_async_remote_copy(
                src_ref=src, dst_ref=dst,
                send_sem=ssems.at[send_slot, c],
                recv_sem=rsems.at[recv_slot, c],
                device_id=dev, device_id_type=_MESH)

        senders = []
        a_in_l, a_in_r = [], []
        b_in = []
        for c in range(_C):
            xchs[c].wait_recv()
            comm_ref[0, c] = p_own[:, c * w:(c + 1) * w] + xrecv_ref[c]
            a_l = copy(comm_ref.at[0, c], comm_ref.at[2, c], left, 0, 1, c)
            a_r = copy(comm_ref.at[0, c], comm_ref.at[1, c], right, 1, 0, c)
            a_l.start()
            a_r.start()
            senders += [a_l, a_r]
            a_in_l.append(copy(comm_ref.at[1, c], comm_ref.at[1, c],
                               left, 0, 0, c))
            a_in_r.append(copy(comm_ref.at[2, c], comm_ref.at[2, c],
                               right, 1, 1, c))
            out_ref[:, pl.ds(r * fq + c * w, w)] = comm_ref[0, c]

        for c in range(_C):
            a_in_l[c].wait_recv()
            b_r = copy(comm_ref.at[1, c, pl.ds(0, hr)],
                       comm_ref.at[3, c, pl.ds(0, hr)], right, 2, 2, c)
            b_r.start()
            senders.append(b_r)
            out_ref[:, pl.ds(lq * fq + c * w, w)] = comm_ref[1, c]
            a_in_r[c].wait_recv()
            b_l = copy(comm_ref.at[2, c, pl.ds(hr, hr)],
                       comm_ref.at[3, c, pl.ds(hr, hr)], left, 3, 3, c)
            b_l.start()
            senders.append(b_l)
            out_ref[:, pl.ds(rq * fq + c * w, w)] = comm_ref[2, c]
            b_in.append(copy(comm_ref.at[3, c, pl.ds(0, hr)],
                             comm_ref.at[3, c, pl.ds(0, hr)], left, 2, 2, c))
            b_in.append(copy(comm_ref.at[3, c, pl.ds(hr, hr)],
                             comm_ref.at[3, c, pl.ds(hr, hr)], right, 3, 3, c))

        for c in range(_C):
            b_in[2 * c].wait_recv()
            b_in[2 * c + 1].wait_recv()
            out_ref[:, pl.ds(dq * fq + c * w, w)] = comm_ref[3, c]
        for snd in senders:
            snd.wait_send()
        for xch in xchs:
            xch.wait_send()

    return pl.pallas_call(
        body,
        out_shape=jax.ShapeDtypeStruct((out_rows, f), jnp.float32),
        in_specs=[pl.BlockSpec(memory_space=pltpu.VMEM),
                  pl.BlockSpec(memory_space=pltpu.VMEM)],
        out_specs=pl.BlockSpec(memory_space=pltpu.VMEM),
        scratch_shapes=[
            pltpu.VMEM((_C, out_rows, w), jnp.float32),
            pltpu.VMEM((_C, out_rows, w), jnp.float32),
            pltpu.VMEM((4, _C, out_rows, w), jnp.float32),
            pltpu.SemaphoreType.DMA((_C,)),
            pltpu.SemaphoreType.DMA((_C,)),
            pltpu.SemaphoreType.DMA((4, _C)),
            pltpu.SemaphoreType.DMA((4, _C)),
        ],
        compiler_params=pltpu.CompilerParams(collective_id=0),
    )(x, dy)


# device time: 21906 ns/iter; 1.7744x vs baseline; 1.0385x over previous
import jax
import jax.numpy as jnp
from jax import lax
from jax.experimental import pallas as pl
from jax.experimental.pallas import tpu as pltpu

_MESH = pl.DeviceIdType.MESH
_C = 4


def kernel(x, dy):
    m, d = x.shape
    _, f = dy.shape
    out_rows = d // 2
    hr = out_rows // 2
    fq = f // 4
    w = fq // _C

    def body(x_ref, dy_ref, out_ref, xsend_ref, xrecv_ref, comm_ref,
             xs_sems, xr_sems, ssems, rsems, ring_bar):
        X = lax.axis_index("x")
        Y = lax.axis_index("y")
        Z = lax.axis_index("z")
        r = 2 * Y + jnp.bitwise_xor(Y, Z)
        lq = jnp.remainder(r + 3, 4)
        rq = jnp.remainder(r + 1, 4)
        dq = jnp.remainder(r + 2, 4)

        def coords(p):
            py = p // 2
            return py, jnp.bitwise_xor(py, jnp.remainder(p, 2))

        ly, lz = coords(lq)
        ry, rz = coords(rq)
        left = (X, ly, lz)
        right = (X, ry, rz)
        xpeer = (1 - X, Y, Z)

        barrier = pltpu.get_barrier_semaphore()
        pl.semaphore_signal(barrier, inc=1, device_id=xpeer,
                            device_id_type=_MESH)
        for nbr in ((X, 1 - Y, Z), (X, Y, 1 - Z)):
            pl.semaphore_signal(ring_bar, inc=1, device_id=nbr,
                                device_id_type=_MESH)

        dims = (((0,), (0,)), ((), ()))
        x_oth = x_ref[:, pl.ds((1 - X) * out_rows, out_rows)]
        x_own = x_ref[:, pl.ds(X * out_rows, out_rows)]

        xchs = []
        for c in range(_C):
            dy_c = dy_ref[:, pl.ds(r * fq + c * w, w)]
            xsend_ref[c] = lax.dot_general(
                x_oth, dy_c, dims, preferred_element_type=jnp.float32)
            if c == 0:
                pl.semaphore_wait(barrier, 1)
            xch = pltpu.make_async_remote_copy(
                src_ref=xsend_ref.at[c], dst_ref=xrecv_ref.at[c],
                send_sem=xs_sems.at[c], recv_sem=xr_sems.at[c],
                device_id=xpeer, device_id_type=_MESH)
            xch.start()
            xchs.append(xch)

        def copy(src, dst, dev, send_slot, recv_slot, c):
            return pltpu.make_async_remote_copy(
                src_ref=src, dst_ref=dst,
                send_sem=ssems.at[send_slot, c],
                recv_sem=rsems.at[recv_slot, c],
                device_id=dev, device_id_type=_MESH)

        senders = []
        a_in_l, a_in_r = [], []
        b_in = []
        for c in range(_C):
            dy_c = dy_ref[:, pl.ds(r * fq + c * w, w)]
            p_own = lax.dot_general(
                x_own, dy_c, dims, preferred_element_type=jnp.float32)
            xchs[c].wait_recv()
            comm_ref[0, c] = p_own + xrecv_ref[c]
            if c == 0:
                pl.semaphore_wait(ring_bar, 2)
            a_l = copy(comm_ref.at[0, c], comm_ref.at[2, c], left, 0, 1, c)
            a_r = copy(comm_ref.at[0, c], comm_ref.at[1, c], right, 1, 0, c)
            a_l.start()
            a_r.start()
            senders += [a_l, a_r]
            a_in_l.append(copy(comm_ref.at[1, c], comm_ref.at[1, c],
                               left, 0, 0, c))
            a_in_r.append(copy(comm_ref.at[2, c], comm_ref.at[2, c],
                               right, 1, 1, c))
            out_ref[:, pl.ds(r * fq + c * w, w)] = comm_ref[0, c]

        for c in range(_C):
            a_in_l[c].wait_recv()
            b_r = copy(comm_ref.at[1, c, pl.ds(0, hr)],
                       comm_ref.at[3, c, pl.ds(0, hr)], right, 2, 2, c)
            b_r.start()
            senders.append(b_r)
            out_ref[:, pl.ds(lq * fq + c * w, w)] = comm_ref[1, c]
            a_in_r[c].wait_recv()
            b_l = copy(comm_ref.at[2, c, pl.ds(hr, hr)],
                       comm_ref.at[3, c, pl.ds(hr, hr)], left, 3, 3, c)
            b_l.start()
            senders.append(b_l)
            out_ref[:, pl.ds(rq * fq + c * w, w)] = comm_ref[2, c]
            b_in.append(copy(comm_ref.at[3, c, pl.ds(0, hr)],
                             comm_ref.at[3, c, pl.ds(0, hr)], left, 2, 2, c))
            b_in.append(copy(comm_ref.at[3, c, pl.ds(hr, hr)],
                             comm_ref.at[3, c, pl.ds(hr, hr)], right, 3, 3, c))

        for c in range(_C):
            b_in[2 * c].wait_recv()
            b_in[2 * c + 1].wait_recv()
            out_ref[:, pl.ds(dq * fq + c * w, w)] = comm_ref[3, c]
        for snd in senders:
            snd.wait_send()
        for xch in xchs:
            xch.wait_send()

    return pl.pallas_call(
        body,
        out_shape=jax.ShapeDtypeStruct((out_rows, f), jnp.float32),
        in_specs=[pl.BlockSpec(memory_space=pltpu.VMEM),
                  pl.BlockSpec(memory_space=pltpu.VMEM)],
        out_specs=pl.BlockSpec(memory_space=pltpu.VMEM),
        scratch_shapes=[
            pltpu.VMEM((_C, out_rows, w), jnp.float32),
            pltpu.VMEM((_C, out_rows, w), jnp.float32),
            pltpu.VMEM((4, _C, out_rows, w), jnp.float32),
            pltpu.SemaphoreType.DMA((_C,)),
            pltpu.SemaphoreType.DMA((_C,)),
            pltpu.SemaphoreType.DMA((4, _C)),
            pltpu.SemaphoreType.DMA((4, _C)),
            pltpu.SemaphoreType.REGULAR,
        ],
        compiler_params=pltpu.CompilerParams(collective_id=0),
    )(x, dy)
